# Initial kernel scaffold; baseline (speedup 1.0000x reference)
import functools

import jax
import jax.numpy as jnp
from jax import lax
from jax.experimental import pallas as pl
from jax.experimental.pallas import tpu as pltpu
from jax.experimental.pallas import tpu_sc as plsc

N = 10000
E = 320000
D_IN = 128
H = 64
EDGE_DIM = 16
OUT = 64
NUM_GRAPHS = 64

NC = 2
NS = 16
NW = NC * NS
EPW = E // NW
C = 80
NCHUNK = EPW // C
ROWS_PER_SUB = N // NS
ZROWS = 125
AW = 80

_mesh = plsc.VectorSubcoreMesh(
    core_axis_name="c", subcore_axis_name="s", num_cores=NC, num_subcores=NS)


def _leaky(x):
    return jnp.where(x >= 0, x, 0.01 * x)


def _ids16(g):
    return g * 16 + lax.iota(jnp.int32, 16)


def _full16(v):
    return jnp.full((16,), v, jnp.int32)


def _sc_edge_pass(gate: bool):
    scratch = [
        pltpu.VMEM((C,), jnp.int32),
        pltpu.VMEM((C,), jnp.int32),
        pltpu.VMEM((C, H), jnp.float32),
        pltpu.VMEM((C, AW), jnp.float32),
        pltpu.VMEM((N,), jnp.float32),
        pltpu.VMEM((N,), jnp.float32),
        pltpu.VMEM((ZROWS, AW), jnp.float32),
        pltpu.VMEM_SHARED((N, AW), jnp.float32),
        pltpu.SemaphoreType.DMA,
        pltpu.SemaphoreType.DMA,
    ]
    if gate:
        scratch += [
            pltpu.VMEM((C, H), jnp.float32),
            pltpu.VMEM((C, H), jnp.float32),
            pltpu.VMEM((C,), jnp.float32),
            pltpu.VMEM((H,), jnp.float32),
        ]

    def body(*refs):
        if gate:
            (src_h, dst_h, qp_h, qgl_h, pap_h, xrow_h, pgl_h, r_h, sgn_h,
             out_h, srcv, dstv, rowsX, msg, tab1, tab2, zb, acc, sem1, sem2,
             rowsP, qv, qglv, sgv) = refs
        else:
            (src_h, dst_h, xrow_h, asrc_h, adst_h,
             out_h, srcv, dstv, rowsX, msg, tab1, tab2, zb, acc, sem1,
             sem2) = refs

        core = lax.axis_index("c")
        sub = lax.axis_index("s")
        wid = core * NS + sub

        if gate:
            pltpu.sync_copy(r_h, tab1)
            pltpu.sync_copy(pgl_h, tab2)
            pltpu.sync_copy(sgn_h, sgv)
        else:
            pltpu.sync_copy(asrc_h, tab1)
            pltpu.sync_copy(adst_h, tab2)

        def zfill(i, _):
            for j in range(AW // 16):
                zb[i, pl.ds(16 * j, 16)] = jnp.zeros((16,), jnp.float32)
            return 0
        lax.fori_loop(0, ZROWS, zfill, 0)
        row0 = sub * ROWS_PER_SUB
        for b in range(ROWS_PER_SUB // ZROWS):
            pltpu.sync_copy(zb, acc.at[pl.ds(row0 + b * ZROWS, ZROWS), :])

        for g in range(C // 16):
            ids = _ids16(g)
            for j in range(H + 1, AW):
                plsc.store_scatter(msg, [ids, _full16(j)],
                                   jnp.zeros((16,), jnp.float32))

        plsc.subcore_barrier()

        def chunk(ci, _):
            base = wid * EPW + ci * C
            pltpu.sync_copy(src_h.at[pl.ds(base, C)], srcv)
            pltpu.sync_copy(dst_h.at[pl.ds(base, C)], dstv)
            cpx = pltpu.async_copy(xrow_h.at[srcv], rowsX, sem1)
            if gate:
                pltpu.sync_copy(qp_h.at[pl.ds(base, C), :], qv)
                pltpu.sync_copy(qgl_h.at[pl.ds(base, C)], qglv)
                cpp = pltpu.async_copy(pap_h.at[srcv], rowsP, sem2)
                cpp.wait()
            cpx.wait()

            def group(g, _):
                ids = _ids16(g)
                src16 = plsc.load_gather(srcv, [ids])
                dst16 = plsc.load_gather(dstv, [ids])
                if gate:
                    accv = jnp.zeros((16,), jnp.float32)
                    for i in range(H):
                        pj = plsc.load_gather(rowsP, [ids, _full16(i)])
                        qj = plsc.load_gather(qv, [ids, _full16(i)])
                        accv = accv + sgv[i] * jnp.abs(pj + qj)
                    pgl16 = plsc.load_gather(tab2, [src16])
                    qgl16 = plsc.load_gather(qglv, [ids])
                    r16 = plsc.load_gather(tab1, [dst16])
                    logit = _leaky(0.505 * (pgl16 + qgl16) + 0.495 * accv
                                   + r16)
                else:
                    as16 = plsc.load_gather(tab1, [src16])
                    ad16 = plsc.load_gather(tab2, [dst16])
                    logit = _leaky(as16 + ad16)
                e16 = jnp.exp(logit)
                for j in range(H):
                    col = plsc.load_gather(rowsX, [ids, _full16(j)])
                    plsc.store_scatter(msg, [ids, _full16(j)], e16 * col)
                plsc.store_scatter(msg, [ids, _full16(H)], e16)
                return 0

            lax.fori_loop(0, C // 16, group, 0)
            pltpu.sync_copy(msg, acc.at[dstv], add=True)
            return 0

        lax.fori_loop(0, NCHUNK, chunk, 0)
        plsc.subcore_barrier()

        for b in range(ROWS_PER_SUB // ZROWS):
            r0 = row0 + b * ZROWS
            pltpu.sync_copy(acc.at[pl.ds(r0, ZROWS), :],
                            out_h.at[core, pl.ds(r0, ZROWS), :])

    return pl.kernel(
        body,
        out_type=jax.ShapeDtypeStruct((NC, N, AW), jnp.float32),
        mesh=_mesh,
        scratch_types=scratch,
    )


_gate_pass = _sc_edge_pass(gate=True)
_gat_pass = _sc_edge_pass(gate=False)


RB = 400
NGRID = N // RB
EB = 512
EGRID = E // EB


def _tc_a1(x, W_pm, b_pm, W_lin1, b_lin1, Ag, agl, gr):
    def body(x_r, wpm_r, bpm_r, wl1_r, bl1_r, ag_r, agl_r, gr_r,
             x2_r, pap_r, pgl_r, r_r):
        x1 = jnp.dot(x_r[...], wpm_r[...],
                     preferred_element_type=jnp.float32) + bpm_r[...]
        x2 = _leaky(jnp.dot(x1, wl1_r[...],
                            preferred_element_type=jnp.float32) + bl1_r[...])
        x2_r[...] = x2
        pap_r[...] = jnp.dot(x2, ag_r[...],
                             preferred_element_type=jnp.float32)
        pgl_r[...] = jnp.dot(x2, agl_r[...],
                             preferred_element_type=jnp.float32)
        r_r[...] = jnp.dot(x2, gr_r[...],
                           preferred_element_type=jnp.float32)

    full = lambda shape: pl.BlockSpec(shape, lambda i: (0, 0))
    row = lambda shape: pl.BlockSpec(shape, lambda i: (i, 0))
    return pl.pallas_call(
        body,
        grid=(NGRID,),
        in_specs=[row((RB, D_IN)), full((D_IN, H)), full((1, H)),
                  full((H, H)), full((1, H)), full((H, H)), full((H, 1)),
                  full((H, 1))],
        out_specs=[row((RB, H)), row((RB, H)), row((RB, 1)), row((RB, 1))],
        out_shape=[jax.ShapeDtypeStruct((N, H), jnp.float32),
                   jax.ShapeDtypeStruct((N, H), jnp.float32),
                   jax.ShapeDtypeStruct((N, 1), jnp.float32),
                   jax.ShapeDtypeStruct((N, 1), jnp.float32)],
    )(x, W_pm, b_pm, W_lin1, b_lin1, Ag, agl, gr)


def _tc_a2(ea, Bg, bgl):
    def body(ea_r, bg_r, bgl_r, qp_r, qgl_r):
        qp_r[...] = jnp.dot(ea_r[...], bg_r[...],
                            preferred_element_type=jnp.float32)
        qgl_r[...] = jnp.dot(ea_r[...], bgl_r[...],
                             preferred_element_type=jnp.float32)

    return pl.pallas_call(
        body,
        grid=(EGRID,),
        in_specs=[pl.BlockSpec((EB, EDGE_DIM), lambda i: (i, 0)),
                  pl.BlockSpec((EDGE_DIM, H), lambda i: (0, 0)),
                  pl.BlockSpec((EDGE_DIM, 1), lambda i: (0, 0))],
        out_specs=[pl.BlockSpec((EB, H), lambda i: (i, 0)),
                   pl.BlockSpec((EB, 1), lambda i: (i, 0))],
        out_shape=[jax.ShapeDtypeStruct((E, H), jnp.float32),
                   jax.ShapeDtypeStruct((E, 1), jnp.float32)],
    )(ea, Bg, bgl)


def _gru_block(hx, hh, Wih, Whh, bih, bhh):
    gi = jnp.dot(hx, Wih, preferred_element_type=jnp.float32) + bih
    gh = jnp.dot(hh, Whh, preferred_element_type=jnp.float32) + bhh
    ir, iz, inn = gi[:, :H], gi[:, H:2 * H], gi[:, 2 * H:]
    hr, hz, hn = gh[:, :H], gh[:, H:2 * H], gh[:, 2 * H:]
    r = jax.nn.sigmoid(ir + hr)
    z = jax.nn.sigmoid(iz + hz)
    nb = jnp.tanh(inn + r * hn)
    return (1.0 - z) * nb + z * hh


def _elu(x):
    return jnp.where(x > 0, x, jnp.exp(jnp.minimum(x, 0.0)) - 1.0)


def _tc_c(acc0, acc1, x2, g_lin2, g_bias, Wih, Whh, bih, bhh,
          gat_W, att_s, att_d):
    def body(a0_r, a1_r, x2_r, gl2_r, gb_r, wih_r, whh_r, bih_r, bhh_r,
             gw_r, ats_r, atd_r, x3_r, xp_r, as_r, ad_r):
        accf = a0_r[...] + a1_r[...]
        s = accf[:, H:H + 1]
        hin = accf[:, :H] / (s + 1e-16)
        h = _elu(jnp.dot(hin, gl2_r[...],
                         preferred_element_type=jnp.float32) + gb_r[...])
        x3 = jnp.maximum(
            _gru_block(h, x2_r[...], wih_r[...], whh_r[...], bih_r[...],
                       bhh_r[...]), 0.0)
        x3_r[...] = x3
        xp = jnp.dot(x3, gw_r[...], preferred_element_type=jnp.float32)
        xp_r[...] = xp
        as_r[...] = jnp.dot(xp, ats_r[...],
                            preferred_element_type=jnp.float32)
        ad_r[...] = jnp.dot(xp, atd_r[...],
                            preferred_element_type=jnp.float32)

    full = lambda shape: pl.BlockSpec(shape, lambda i: (0, 0))
    row = lambda shape: pl.BlockSpec(shape, lambda i: (i, 0))
    return pl.pallas_call(
        body,
        grid=(NGRID,),
        in_specs=[row((RB, AW)), row((RB, AW)), row((RB, H)),
                  full((H, H)), full((1, H)), full((H, 3 * H)),
                  full((H, 3 * H)), full((1, 3 * H)), full((1, 3 * H)),
                  full((H, H)), full((H, 1)), full((H, 1))],
        out_specs=[row((RB, H)), row((RB, H)), row((RB, 1)), row((RB, 1))],
        out_shape=[jax.ShapeDtypeStruct((N, H), jnp.float32),
                   jax.ShapeDtypeStruct((N, H), jnp.float32),
                   jax.ShapeDtypeStruct((N, 1), jnp.float32),
                   jax.ShapeDtypeStruct((N, 1), jnp.float32)],
    )(acc0, acc1, x2, g_lin2, g_bias, Wih, Whh, bih, bhh, gat_W,
      att_s, att_d)


def _tc_e(acc0, acc1, x3, batchf, gat_bias, Wih, Whh, bih, bhh,
          W_lin2, b_lin2):
    def body(a0_r, a1_r, x3_r, b_r, gb_r, wih_r, whh_r, bih_r, bhh_r,
             wl2_r, bl2_r, out_r):
        accf = a0_r[...] + a1_r[...]
        s = accf[:, H:H + 1]
        h2 = _elu(accf[:, :H] / (s + 1e-16) + gb_r[...])
        x4 = jnp.maximum(
            _gru_block(h2, x3_r[...], wih_r[...], whh_r[...], bih_r[...],
                       bhh_r[...]), 0.0)
        node = jnp.dot(x4, wl2_r[...],
                       preferred_element_type=jnp.float32) + bl2_r[...]
        gid = lax.broadcasted_iota(jnp.float32, (RB, NUM_GRAPHS), 1)
        onehot = (b_r[...] == gid).astype(jnp.float32)
        contrib = lax.dot_general(onehot, node, (((0,), (0,)), ((), ())),
                                  preferred_element_type=jnp.float32)

        @pl.when(pl.program_id(0) == 0)
        def _():
            out_r[...] = jnp.zeros((NUM_GRAPHS, OUT), jnp.float32)

        out_r[...] += contrib

    full = lambda shape: pl.BlockSpec(shape, lambda i: (0, 0))
    row = lambda shape: pl.BlockSpec(shape, lambda i: (i, 0))
    return pl.pallas_call(
        body,
        grid=(NGRID,),
        in_specs=[row((RB, AW)), row((RB, AW)), row((RB, H)), row((RB, 1)),
                  full((1, H)), full((H, 3 * H)), full((H, 3 * H)),
                  full((1, 3 * H)), full((1, 3 * H)), full((H, OUT)),
                  full((1, OUT))],
        out_specs=pl.BlockSpec((NUM_GRAPHS, OUT), lambda i: (0, 0)),
        out_shape=jax.ShapeDtypeStruct((NUM_GRAPHS, OUT), jnp.float32),
    )(acc0, acc1, x3, batchf, gat_bias, Wih, Whh, bih, bhh, W_lin2, b_lin2)


def kernel(x, edge_index, edge_attr, batch, W_pm, b_pm, W_lin1, b_lin1,
           g_lin1, g_lin2, g_att_l, g_att_r, g_bias, gru0_Wih, gru0_Whh,
           gru0_bih, gru0_bhh, gat_W, gat_att_src, gat_att_dst, gat_bias,
           gru1_Wih, gru1_Whh, gru1_bih, gru1_bhh, W_lin2, b_lin2):
    src = edge_index[0]
    dst = edge_index[1]
    gl = g_att_l
    A = g_lin1[:H]
    B = g_lin1[H:]
    Ag = A * gl[None, :]
    Bg = B * gl[None, :]
    agl = (A @ gl).reshape(H, 1)
    bgl = (B @ gl).reshape(EDGE_DIM, 1)
    sgn = jnp.sign(gl)

    x2, pAp, pgl, r = _tc_a1(x, W_pm, b_pm.reshape(1, H), W_lin1,
                             b_lin1.reshape(1, H), Ag, agl,
                             g_att_r.reshape(H, 1))
    Qp, qgl = _tc_a2(edge_attr, Bg, bgl)

    acc = _gate_pass(src, dst, Qp, qgl.reshape(E), pAp, x2,
                     pgl.reshape(N), r.reshape(N), sgn)

    x3, xp, asrc, adst = _tc_c(acc[0], acc[1], x2, g_lin2,
                               g_bias.reshape(1, H), gru0_Wih, gru0_Whh,
                               gru0_bih.reshape(1, 3 * H),
                               gru0_bhh.reshape(1, 3 * H), gat_W,
                               gat_att_src.reshape(H, 1),
                               gat_att_dst.reshape(H, 1))

    acc2 = _gat_pass(src, dst, xp, asrc.reshape(N), adst.reshape(N))

    batchf = batch.astype(jnp.float32).reshape(N, 1)
    graph = _tc_e(acc2[0], acc2[1], x3, batchf, gat_bias.reshape(1, H),
                  gru1_Wih, gru1_Whh, gru1_bih.reshape(1, 3 * H),
                  gru1_bhh.reshape(1, 3 * H), W_lin2,
                  b_lin2.reshape(1, OUT))
    return graph


# speedup vs baseline: 10.6082x; 10.6082x over previous
import functools

import jax
import jax.numpy as jnp
from jax import lax
from jax.experimental import pallas as pl
from jax.experimental.pallas import tpu as pltpu
from jax.experimental.pallas import tpu_sc as plsc

N = 10000
NP = 10240
E = 320000
D_IN = 128
H = 64
EDGE_DIM = 16
OUT = 64
NUM_GRAPHS = 64

NC = 2
NS = 16
NW = NC * NS
EPW = E // NW
C = 80
NCHUNK = EPW // C
ROWS_PER_SUB = NP // NS
ZROWS = 128
AW = 80
TW = 128


def _sc_params():
    return pltpu.CompilerParams(needs_layout_passes=False,
                                use_tc_tiling_on_sc=False)

@functools.lru_cache(maxsize=None)
def _mesh():
    return plsc.VectorSubcoreMesh(
        core_axis_name="c", subcore_axis_name="s",
        num_cores=NC, num_subcores=NS)


def _leaky(x):
    return jnp.where(x >= 0, x, 0.01 * x)


def _ids16(g):
    return g * 16 + lax.iota(jnp.int32, 16)


def _full16(v):
    return jnp.full((16,), v, jnp.int32)


def _sc_edge_pass(gate: bool):
    C = 80 if gate else 128
    TW = 128 if gate else 64
    UPW = 137
    NCH = -(-EPW // C)
    assert NCH % 2 == 1
    nsem = 10 if gate else 8
    scratch = ([pltpu.VMEM((C,), jnp.int32)] * 6
               + [pltpu.VMEM((C, TW), jnp.float32)] * 2
               + [pltpu.VMEM((C, AW), jnp.float32)] * 2
               + [pltpu.VMEM((NP,), jnp.float32)]
               + ([pltpu.VMEM((C, TW), jnp.float32)] * 2
                  + [pltpu.VMEM((C, UPW), jnp.float32),
                     pltpu.VMEM((H,), jnp.float32)]
                  if gate else
                  [pltpu.VMEM((NP,), jnp.float32)])
               + [pltpu.VMEM_SHARED((NP, AW), jnp.float32)]
               + [pltpu.SemaphoreType.DMA] * nsem)

    def body(*refs):
        if gate:
            (src_h, dst_h, qm_h, t_h, r_h, sgn_h, out_h,
             srcv0, srcv1, dstv0, dstv1, dsts0, dsts1, rows0, rows1,
             msg0, msg1, tab1, qv0, qv1, upv, sgv, acc,
             semg0, semg1, semis0, semis1, semid0, semid1,
             sems0, sems1, semq0, semq1) = refs
            tab2 = None
        else:
            (src_h, dst_h, t_h, asrc_h, adst_h, out_h,
             srcv0, srcv1, dstv0, dstv1, dsts0, dsts1, rows0, rows1,
             msg0, msg1, tab1, tab2, acc,
             semg0, semg1, semis0, semis1, semid0, semid1,
             sems0, sems1) = refs
            qv0 = qv1 = upv = sgv = semq0 = semq1 = None
        srcv = (srcv0, srcv1)
        dstv = (dstv0, dstv1)
        dsts = (dsts0, dsts1)
        rows = (rows0, rows1)
        msgs = (msg0, msg1)
        qv = (qv0, qv1)
        semg = (semg0, semg1)
        semis = (semis0, semis1)
        semid = (semid0, semid1)
        sems = (sems0, sems1)
        semq = (semq0, semq1)

        core = lax.axis_index("c")
        sub = lax.axis_index("s")
        wid = core * NS + sub

        if gate:
            pltpu.sync_copy(r_h, tab1)
            pltpu.sync_copy(sgn_h, sgv)
        else:
            pltpu.sync_copy(asrc_h, tab1)
            pltpu.sync_copy(adst_h, tab2)

        def zfill(i, _):
            for m in msgs:
                for j in range(AW // 16):
                    m[i, pl.ds(16 * j, 16)] = jnp.zeros((16,), jnp.float32)
            return 0
        lax.fori_loop(0, C, zfill, 0)
        row0 = sub * ROWS_PER_SUB
        nz = -(-ROWS_PER_SUB // C)
        for b in range(nz):
            rz = row0 + min(b * C, ROWS_PER_SUB - C)
            pltpu.sync_copy(msgs[0], acc.at[pl.ds(rz, C), :])

        plsc.subcore_barrier()

        def lbase(c):
            return jnp.minimum(c * C, EPW - C)

        def base_of(c):
            return wid * EPW + lbase(c)

        def issue_rowload(c, s):
            pltpu.async_copy(t_h.at[srcv[s]], rows[s], semg[s])
            if gate:
                pltpu.async_copy(qm_h.at[pl.ds(base_of(c), C), :TW], qv[s],
                                 semq[s])

        def wait_rows(s):
            pltpu.make_async_copy(t_h.at[pl.ds(0, C), :TW], rows[s],
                                  semg[s]).wait()
            if gate:
                pltpu.make_async_copy(qm_h.at[pl.ds(0, C), :TW], qv[s],
                                      semq[s]).wait()

        def wait_idx(s):
            pltpu.make_async_copy(src_h.at[pl.ds(0, C)], srcv[s],
                                  semis[s]).wait()
            pltpu.make_async_copy(dst_h.at[pl.ds(0, C)], dstv[s],
                                  semid[s]).wait()

        def wait_scatter(s):
            pltpu.make_async_copy(out_h.at[0, pl.ds(0, C), :], msgs[s],
                                  sems[s]).wait()

        def compute(c, s):
            msg = msgs[s]
            thresh = c * C - lbase(c)
            iota16 = lax.iota(jnp.int32, 16)

            def group(g, _):
                ids = g * 16 + iota16
                dst16 = plsc.load_gather(dstv[s], [ids])
                if gate:
                    for k in range(16):
                        rk = _full16(1) * 0 + (ids[k])
                        for j in range(H // 16):
                            cj = 16 * j + iota16
                            pj = plsc.load_gather(rows[s], [rk, H + cj])
                            qj = plsc.load_gather(qv[s], [rk, cj])
                            plsc.store_scatter(upv, [rk, cj], pj + qj)
                    sg = [sgv[pl.ds(16 * k, 16)] for k in range(H // 16)]
                    av = [jnp.zeros((16,), jnp.float32) for _ in range(4)]
                    lv = [jnp.zeros((16,), jnp.float32) for _ in range(4)]
                    for i in range(H):
                        u = plsc.load_gather(upv, [ids, _full16(i)])
                        lv[i % 4] = lv[i % 4] + u
                        av[i % 4] = (av[i % 4]
                                     + sg[i // 16][i % 16] * jnp.abs(u))
                    accv = (av[0] + av[1]) + (av[2] + av[3])
                    linv = (lv[0] + lv[1]) + (lv[2] + lv[3])
                    r16 = plsc.load_gather(tab1, [dst16])
                    logit = _leaky(0.505 * linv + 0.495 * accv + r16)
                else:
                    src16 = plsc.load_gather(srcv[s], [ids])
                    as16 = plsc.load_gather(tab1, [src16])
                    ad16 = plsc.load_gather(tab2, [dst16])
                    logit = _leaky(as16 + ad16)
                e16 = jnp.exp(logit)
                e16 = jnp.where(ids >= thresh, e16, 0.0)
                for k in range(16):
                    rk = _full16(1) * 0 + (ids[k])
                    ek = e16[k]
                    for j in range(H // 16):
                        cj = 16 * j + iota16
                        col = plsc.load_gather(rows[s], [rk, cj])
                        plsc.store_scatter(msg, [rk, cj], ek * col)
                    ecol = jnp.where(iota16 == 0, ek, 0.0)
                    plsc.store_scatter(msg, [rk, H + iota16], ecol)
                dsts[s][pl.ds(g * 16, 16)] = dst16
                return 0

            lax.fori_loop(0, C // 16, group, 0)
            pltpu.async_copy(msg, acc.at[dsts[s]], sems[s], add=True)

        pltpu.sync_copy(src_h.at[pl.ds(base_of(0), C)], srcv[0])
        pltpu.sync_copy(dst_h.at[pl.ds(base_of(0), C)], dstv[0])
        issue_rowload(0, 0)
        pltpu.async_copy(src_h.at[pl.ds(base_of(1), C)], srcv[1], semis[1])
        pltpu.async_copy(dst_h.at[pl.ds(base_of(1), C)], dstv[1], semid[1])

        def section(c, s):
            ns = 1 - s
            wait_rows(s)
            wait_idx(ns)
            issue_rowload(c + 1, ns)
            pltpu.async_copy(src_h.at[pl.ds(base_of(c + 2), C)], srcv[s],
                             semis[s])
            wait_scatter(s)
            compute(c, s)
            pltpu.async_copy(dst_h.at[pl.ds(base_of(c + 2), C)], dstv[s],
                             semid[s])

        def pair(k, _):
            section(2 * k, 0)
            section(2 * k + 1, 1)
            return 0

        for s in (0, 1):
            dv = dsts[s]

            def zidx(i, _):
                dv[pl.ds(i * 16, 16)] = jnp.zeros((16,), jnp.int32)
                return 0
            lax.fori_loop(0, C // 16, zidx, 0)
            pltpu.async_copy(msgs[s], acc.at[dsts[s]], sems[s], add=True)

        lax.fori_loop(0, (NCH - 1) // 2, pair, 0)

        wait_rows(0)
        wait_idx(1)
        wait_scatter(0)
        compute(NCH - 1, 0)
        wait_scatter(0)
        wait_scatter(1)

        plsc.subcore_barrier()
        for b in range(ROWS_PER_SUB // ZROWS):
            r0 = row0 + b * ZROWS
            pltpu.sync_copy(acc.at[pl.ds(r0, ZROWS), :],
                            out_h.at[core, pl.ds(r0, ZROWS), :])

    return pl.kernel(
        body,
        out_type=jax.ShapeDtypeStruct((NC, NP, AW), jnp.float32),
        mesh=_mesh(),
        compiler_params=_sc_params(),
        scratch_types=scratch,
    )


_sc_edge_pass = functools.lru_cache(maxsize=None)(_sc_edge_pass)


RB = 512
NGRID = NP // RB
EB = 512
EGRID = E // EB


def _tc_a1(x, W_pm, b_pm, W_lin1, b_lin1, Ag, gr):
    def body(x_r, wpm_r, bpm_r, wl1_r, bl1_r, ag_r, gr_r,
             t_r, x2_r, r_r):
        x1 = jnp.dot(x_r[...], wpm_r[...],
                     preferred_element_type=jnp.float32) + bpm_r[...]
        x2 = _leaky(jnp.dot(x1, wl1_r[...],
                            preferred_element_type=jnp.float32) + bl1_r[...])
        x2_r[...] = x2
        pap = jnp.dot(x2, ag_r[...], preferred_element_type=jnp.float32)
        t_r[...] = jnp.concatenate([x2, pap], axis=1)
        r_r[...] = jnp.dot(x2, gr_r[...],
                           preferred_element_type=jnp.float32)

    full = lambda shape: pl.BlockSpec(shape, lambda i: (0, 0))
    row = lambda shape: pl.BlockSpec(shape, lambda i: (i, 0))
    return pl.pallas_call(
        body,
        grid=(NGRID,),
        in_specs=[row((RB, D_IN)), full((D_IN, H)), full((1, H)),
                  full((H, H)), full((1, H)), full((H, H)), full((H, 1))],
        out_specs=[row((RB, 2 * H)), row((RB, H)), row((RB, 1))],
        out_shape=[jax.ShapeDtypeStruct((NP, 2 * H), jnp.float32),
                   jax.ShapeDtypeStruct((NP, H), jnp.float32),
                   jax.ShapeDtypeStruct((NP, 1), jnp.float32)],
    )(x, W_pm, b_pm, W_lin1, b_lin1, Ag, gr)


def _tc_a2(ea, Bg):
    def body(ea_r, bg_r, qp_r):
        q = jnp.dot(ea_r[...], bg_r[...],
                    preferred_element_type=jnp.float32)
        qp_r[...] = jnp.concatenate(
            [q, jnp.zeros((EB, TW - H), jnp.float32)], axis=1)

    return pl.pallas_call(
        body,
        grid=(EGRID,),
        in_specs=[pl.BlockSpec((EB, EDGE_DIM), lambda i: (i, 0)),
                  pl.BlockSpec((EDGE_DIM, H), lambda i: (0, 0))],
        out_specs=pl.BlockSpec((EB, TW), lambda i: (i, 0)),
        out_shape=jax.ShapeDtypeStruct((E, TW), jnp.float32),
    )(ea, Bg)


def _gru_block(hx, hh, Wih, Whh, bih, bhh):
    gi = jnp.dot(hx, Wih, preferred_element_type=jnp.float32) + bih
    gh = jnp.dot(hh, Whh, preferred_element_type=jnp.float32) + bhh
    ir, iz, inn = gi[:, :H], gi[:, H:2 * H], gi[:, 2 * H:]
    hr, hz, hn = gh[:, :H], gh[:, H:2 * H], gh[:, 2 * H:]
    r = jax.nn.sigmoid(ir + hr)
    z = jax.nn.sigmoid(iz + hz)
    nb = jnp.tanh(inn + r * hn)
    return (1.0 - z) * nb + z * hh


def _elu(x):
    return jnp.where(x > 0, x, jnp.exp(jnp.minimum(x, 0.0)) - 1.0)


def _tc_c(accb, x2, g_lin2, g_bias, Wih, Whh, bih, bhh,
          gat_W, att_s, att_d):
    def body(a_r, x2_r, gl2_r, gb_r, wih_r, whh_r, bih_r, bhh_r,
             gw_r, ats_r, atd_r, x3_r, xp_r, as_r, ad_r):
        accf = a_r[0] + a_r[1]
        s = accf[:, H:H + 1]
        hin = accf[:, :H] / (s + 1e-16)
        h = _elu(jnp.dot(hin, gl2_r[...],
                         preferred_element_type=jnp.float32) + gb_r[...])
        x3 = jnp.maximum(
            _gru_block(h, x2_r[...], wih_r[...], whh_r[...], bih_r[...],
                       bhh_r[...]), 0.0)
        x3_r[...] = x3
        xp = jnp.dot(x3, gw_r[...], preferred_element_type=jnp.float32)
        xp_r[...] = xp
        as_r[...] = jnp.dot(xp, ats_r[...],
                            preferred_element_type=jnp.float32)
        ad_r[...] = jnp.dot(xp, atd_r[...],
                            preferred_element_type=jnp.float32)

    full = lambda shape: pl.BlockSpec(shape, lambda i: (0, 0))
    row = lambda shape: pl.BlockSpec(shape, lambda i: (i, 0))
    return pl.pallas_call(
        body,
        grid=(NGRID,),
        in_specs=[pl.BlockSpec((NC, RB, AW), lambda i: (0, i, 0)),
                  row((RB, H)),
                  full((H, H)), full((1, H)), full((H, 3 * H)),
                  full((H, 3 * H)), full((1, 3 * H)), full((1, 3 * H)),
                  full((H, H)), full((H, 1)), full((H, 1))],
        out_specs=[row((RB, H)), row((RB, H)), row((RB, 1)),
                   row((RB, 1))],
        out_shape=[jax.ShapeDtypeStruct((NP, H), jnp.float32),
                   jax.ShapeDtypeStruct((NP, H), jnp.float32),
                   jax.ShapeDtypeStruct((NP, 1), jnp.float32),
                   jax.ShapeDtypeStruct((NP, 1), jnp.float32)],
    )(accb, x2, g_lin2, g_bias, Wih, Whh, bih, bhh, gat_W,
      att_s, att_d)


def _tc_e(accb, x3, batchf, gat_bias, Wih, Whh, bih, bhh,
          W_lin2, b_lin2):
    def body(a_r, x3_r, b_r, gb_r, wih_r, whh_r, bih_r, bhh_r,
             wl2_r, bl2_r, out_r):
        accf = a_r[0] + a_r[1]
        s = accf[:, H:H + 1]
        h2 = _elu(accf[:, :H] / (s + 1e-16) + gb_r[...])
        x4 = jnp.maximum(
            _gru_block(h2, x3_r[...], wih_r[...], whh_r[...], bih_r[...],
                       bhh_r[...]), 0.0)
        node = jnp.dot(x4, wl2_r[...],
                       preferred_element_type=jnp.float32) + bl2_r[...]
        gid = lax.broadcasted_iota(jnp.int32, (RB, NUM_GRAPHS),
                                   1).astype(jnp.float32)
        onehot = (b_r[...] == gid).astype(jnp.float32)
        contrib = lax.dot_general(onehot, node, (((0,), (0,)), ((), ())),
                                  preferred_element_type=jnp.float32)

        @pl.when(pl.program_id(0) == 0)
        def _():
            out_r[...] = jnp.zeros((NUM_GRAPHS, OUT), jnp.float32)

        out_r[...] += contrib

    full = lambda shape: pl.BlockSpec(shape, lambda i: (0, 0))
    row = lambda shape: pl.BlockSpec(shape, lambda i: (i, 0))
    return pl.pallas_call(
        body,
        grid=(NGRID,),
        in_specs=[pl.BlockSpec((NC, RB, AW), lambda i: (0, i, 0)),
                  row((RB, H)), row((RB, 1)),
                  full((1, H)), full((H, 3 * H)), full((H, 3 * H)),
                  full((1, 3 * H)), full((1, 3 * H)), full((H, OUT)),
                  full((1, OUT))],
        out_specs=pl.BlockSpec((NUM_GRAPHS, OUT), lambda i: (0, 0)),
        out_shape=jax.ShapeDtypeStruct((NUM_GRAPHS, OUT), jnp.float32),
    )(accb, x3, batchf, gat_bias, Wih, Whh, bih, bhh, W_lin2, b_lin2)


def kernel(x, edge_index, edge_attr, batch, W_pm, b_pm, W_lin1, b_lin1,
           g_lin1, g_lin2, g_att_l, g_att_r, g_bias, gru0_Wih, gru0_Whh,
           gru0_bih, gru0_bhh, gat_W, gat_att_src, gat_att_dst, gat_bias,
           gru1_Wih, gru1_Whh, gru1_bih, gru1_bhh, W_lin2, b_lin2):
    src = edge_index[0]
    dst = edge_index[1]
    gl = g_att_l
    A = g_lin1[:H]
    B = g_lin1[H:]
    Ag = A * gl[None, :]
    Bg = B * gl[None, :]
    sgn = jnp.sign(gl)

    xpad = jnp.pad(x, ((0, NP - N), (0, 0)))
    T, x2, r = _tc_a1(xpad, W_pm, b_pm.reshape(1, H), W_lin1,
                      b_lin1.reshape(1, H), Ag, g_att_r.reshape(H, 1))
    Qp = _tc_a2(edge_attr, Bg)

    acc = _sc_edge_pass(True)(src, dst, Qp, T, r.reshape(NP), sgn)

    x3, xp, asrc, adst = _tc_c(acc, x2, g_lin2,
                               g_bias.reshape(1, H), gru0_Wih, gru0_Whh,
                               gru0_bih.reshape(1, 3 * H),
                               gru0_bhh.reshape(1, 3 * H), gat_W,
                               gat_att_src.reshape(H, 1),
                               gat_att_dst.reshape(H, 1))

    acc2 = _sc_edge_pass(False)(src, dst, xp, asrc.reshape(NP),
                                adst.reshape(NP))

    batchf = jnp.pad(batch.astype(jnp.float32), (0, NP - N),
                     constant_values=-1.0).reshape(NP, 1)
    graph = _tc_e(acc2, x3, batchf, gat_bias.reshape(1, H),
                  gru1_Wih, gru1_Whh, gru1_bih.reshape(1, 3 * H),
                  gru1_bhh.reshape(1, 3 * H), W_lin2,
                  b_lin2.reshape(1, OUT))
    return graph


# speedup vs baseline: 10.6650x; 1.0054x over previous
import functools

import jax
import jax.numpy as jnp
from jax import lax
from jax.experimental import pallas as pl
from jax.experimental.pallas import tpu as pltpu
from jax.experimental.pallas import tpu_sc as plsc

N = 10000
NP = 10240
E = 320000
D_IN = 128
H = 64
EDGE_DIM = 16
OUT = 64
NUM_GRAPHS = 64

NC = 2
NS = 16
NW = NC * NS
EPW = E // NW
C = 80
NCHUNK = EPW // C
ROWS_PER_SUB = NP // NS
ZROWS = 128
AW = 80
TW = 128


def _sc_params():
    return pltpu.CompilerParams(needs_layout_passes=False,
                                use_tc_tiling_on_sc=False)

@functools.lru_cache(maxsize=None)
def _mesh():
    return plsc.VectorSubcoreMesh(
        core_axis_name="c", subcore_axis_name="s",
        num_cores=NC, num_subcores=NS)


def _leaky(x):
    return jnp.where(x >= 0, x, 0.01 * x)


def _ids16(g):
    return g * 16 + lax.iota(jnp.int32, 16)


def _full16(v):
    return jnp.full((16,), v, jnp.int32)


def _sc_edge_pass(gate: bool):
    C = 80 if gate else 128
    TW = 128 if gate else 64
    UPW = 137
    NCH = -(-EPW // C)
    assert NCH % 2 == 1
    nsem = 10 if gate else 8
    scratch = ([pltpu.VMEM((C,), jnp.int32)] * 6
               + [pltpu.VMEM((C, TW), jnp.float32)] * 2
               + [pltpu.VMEM((C, AW), jnp.float32)] * 2
               + [pltpu.VMEM((NP,), jnp.float32)]
               + ([pltpu.VMEM((C, TW), jnp.float32)] * 2
                  + [pltpu.VMEM((C, UPW), jnp.float32),
                     pltpu.VMEM((H,), jnp.float32)]
                  if gate else
                  [pltpu.VMEM((NP,), jnp.float32)])
               + [pltpu.VMEM_SHARED((NP, AW), jnp.float32)]
               + [pltpu.SemaphoreType.DMA] * nsem)

    def body(*refs):
        if gate:
            (src_h, dst_h, qm_h, t_h, r_h, sgn_h, out_h,
             srcv0, srcv1, dstv0, dstv1, dsts0, dsts1, rows0, rows1,
             msg0, msg1, tab1, qv0, qv1, upv, sgv, acc,
             semg0, semg1, semis0, semis1, semid0, semid1,
             sems0, sems1, semq0, semq1) = refs
            tab2 = None
        else:
            (src_h, dst_h, t_h, asrc_h, adst_h, out_h,
             srcv0, srcv1, dstv0, dstv1, dsts0, dsts1, rows0, rows1,
             msg0, msg1, tab1, tab2, acc,
             semg0, semg1, semis0, semis1, semid0, semid1,
             sems0, sems1) = refs
            qv0 = qv1 = upv = sgv = semq0 = semq1 = None
        srcv = (srcv0, srcv1)
        dstv = (dstv0, dstv1)
        dsts = (dsts0, dsts1)
        rows = (rows0, rows1)
        msgs = (msg0, msg1)
        qv = (qv0, qv1)
        semg = (semg0, semg1)
        semis = (semis0, semis1)
        semid = (semid0, semid1)
        sems = (sems0, sems1)
        semq = (semq0, semq1)

        core = lax.axis_index("c")
        sub = lax.axis_index("s")
        wid = core * NS + sub

        if gate:
            pltpu.sync_copy(r_h, tab1)
            pltpu.sync_copy(sgn_h, sgv)
        else:
            pltpu.sync_copy(asrc_h, tab1)
            pltpu.sync_copy(adst_h, tab2)

        def zfill(i, _):
            for m in msgs:
                for j in range(AW // 16):
                    m[i, pl.ds(16 * j, 16)] = jnp.zeros((16,), jnp.float32)
            return 0
        lax.fori_loop(0, C, zfill, 0)
        row0 = sub * ROWS_PER_SUB
        nz = -(-ROWS_PER_SUB // C)
        for b in range(nz):
            rz = row0 + min(b * C, ROWS_PER_SUB - C)
            pltpu.sync_copy(msgs[0], acc.at[pl.ds(rz, C), :])

        plsc.subcore_barrier()

        def lbase(c):
            return jnp.minimum(c * C, EPW - C)

        def base_of(c):
            return wid * EPW + lbase(c)

        def issue_rowload(c, s):
            pltpu.async_copy(t_h.at[srcv[s]], rows[s], semg[s])
            if gate:
                pltpu.async_copy(qm_h.at[pl.ds(base_of(c), C), :TW], qv[s],
                                 semq[s])

        def wait_rows(s):
            pltpu.make_async_copy(t_h.at[pl.ds(0, C), :TW], rows[s],
                                  semg[s]).wait()
            if gate:
                pltpu.make_async_copy(qm_h.at[pl.ds(0, C), :TW], qv[s],
                                      semq[s]).wait()

        def wait_idx(s):
            pltpu.make_async_copy(src_h.at[pl.ds(0, C)], srcv[s],
                                  semis[s]).wait()
            pltpu.make_async_copy(dst_h.at[pl.ds(0, C)], dstv[s],
                                  semid[s]).wait()

        def wait_scatter(s):
            pltpu.make_async_copy(out_h.at[0, pl.ds(0, C), :], msgs[s],
                                  sems[s]).wait()

        def compute(c, s):
            msg = msgs[s]
            thresh = c * C - lbase(c)
            iota16 = lax.iota(jnp.int32, 16)

            def group(g, _):
                ids = g * 16 + iota16
                dst16 = plsc.load_gather(dstv[s], [ids])
                if gate:
                    for k in range(16):
                        rk = _full16(1) * 0 + (ids[k])
                        for j in range(H // 16):
                            cj = 16 * j + iota16
                            pj = plsc.load_gather(rows[s], [rk, H + cj])
                            qj = plsc.load_gather(qv[s], [rk, cj])
                            plsc.store_scatter(upv, [rk, cj], pj + qj)
                    sg = [sgv[pl.ds(16 * k, 16)] for k in range(H // 16)]
                    av = [jnp.zeros((16,), jnp.float32) for _ in range(4)]
                    lv = [jnp.zeros((16,), jnp.float32) for _ in range(4)]
                    for i in range(H):
                        u = plsc.load_gather(upv, [ids, _full16(i)])
                        lv[i % 4] = lv[i % 4] + u
                        av[i % 4] = (av[i % 4]
                                     + sg[i // 16][i % 16] * jnp.abs(u))
                    accv = (av[0] + av[1]) + (av[2] + av[3])
                    linv = (lv[0] + lv[1]) + (lv[2] + lv[3])
                    r16 = plsc.load_gather(tab1, [dst16])
                    logit = _leaky(0.505 * linv + 0.495 * accv + r16)
                else:
                    src16 = plsc.load_gather(srcv[s], [ids])
                    as16 = plsc.load_gather(tab1, [src16])
                    ad16 = plsc.load_gather(tab2, [dst16])
                    logit = _leaky(as16 + ad16)
                e16 = jnp.exp(logit)
                e16 = jnp.where(ids >= thresh, e16, 0.0)
                for k in range(16):
                    rk = _full16(1) * 0 + (ids[k])
                    ek = e16[k]
                    for j in range(H // 16):
                        cj = 16 * j + iota16
                        col = plsc.load_gather(rows[s], [rk, cj])
                        plsc.store_scatter(msg, [rk, cj], ek * col)
                    ecol = jnp.where(iota16 == 0, ek, 0.0)
                    plsc.store_scatter(msg, [rk, H + iota16], ecol)
                dsts[s][pl.ds(g * 16, 16)] = dst16
                return 0

            lax.fori_loop(0, C // 16, group, 0)
            pltpu.async_copy(msg, acc.at[dsts[s]], sems[s], add=True)

        pltpu.sync_copy(src_h.at[pl.ds(base_of(0), C)], srcv[0])
        pltpu.sync_copy(dst_h.at[pl.ds(base_of(0), C)], dstv[0])
        issue_rowload(0, 0)
        pltpu.async_copy(src_h.at[pl.ds(base_of(1), C)], srcv[1], semis[1])
        pltpu.async_copy(dst_h.at[pl.ds(base_of(1), C)], dstv[1], semid[1])

        def section(c, s):
            ns = 1 - s
            wait_rows(s)
            wait_idx(ns)
            issue_rowload(c + 1, ns)
            pltpu.async_copy(src_h.at[pl.ds(base_of(c + 2), C)], srcv[s],
                             semis[s])
            wait_scatter(s)
            compute(c, s)
            pltpu.async_copy(dst_h.at[pl.ds(base_of(c + 2), C)], dstv[s],
                             semid[s])

        def pair(k, _):
            section(2 * k, 0)
            section(2 * k + 1, 1)
            return 0

        for s in (0, 1):
            dv = dsts[s]

            def zidx(i, _):
                dv[pl.ds(i * 16, 16)] = jnp.zeros((16,), jnp.int32)
                return 0
            lax.fori_loop(0, C // 16, zidx, 0)
            pltpu.async_copy(msgs[s], acc.at[dsts[s]], sems[s], add=True)

        lax.fori_loop(0, (NCH - 1) // 2, pair, 0)

        wait_rows(0)
        wait_idx(1)
        wait_scatter(0)
        compute(NCH - 1, 0)
        wait_scatter(0)
        wait_scatter(1)

        plsc.subcore_barrier()
        for b in range(ROWS_PER_SUB // ZROWS):
            r0 = row0 + b * ZROWS
            pltpu.sync_copy(acc.at[pl.ds(r0, ZROWS), :],
                            out_h.at[core, pl.ds(r0, ZROWS), :])

    return pl.kernel(
        body,
        out_type=jax.ShapeDtypeStruct((NC, NP, AW), jnp.float32),
        mesh=_mesh(),
        compiler_params=_sc_params(),
        scratch_types=scratch,
    )


_sc_edge_pass = functools.lru_cache(maxsize=None)(_sc_edge_pass)


RB = 512
NGRID = NP // RB
EB = 512
EGRID = E // EB
TW0 = 128


def _tc_a(x, W_pm, b_pm, W_lin1, b_lin1, Ag, gr, ea, Bg):
    def body(x_r, wpm_r, bpm_r, wl1_r, bl1_r, ag_r, gr_r, ea_r, bg_r,
             t_r, x2_r, r_r, qp_r):
        q = jnp.dot(ea_r[...], bg_r[...],
                    preferred_element_type=jnp.float32)
        qp_r[...] = jnp.concatenate(
            [q, jnp.zeros((EB, TW0 - H), jnp.float32)], axis=1)

        @pl.when(pl.program_id(0) < NGRID)
        def _():
            x1 = jnp.dot(x_r[...], wpm_r[...],
                         preferred_element_type=jnp.float32) + bpm_r[...]
            x2 = _leaky(jnp.dot(x1, wl1_r[...],
                                preferred_element_type=jnp.float32)
                        + bl1_r[...])
            x2_r[...] = x2
            pap = jnp.dot(x2, ag_r[...], preferred_element_type=jnp.float32)
            t_r[...] = jnp.concatenate([x2, pap], axis=1)
            r_r[...] = jnp.dot(x2, gr_r[...],
                               preferred_element_type=jnp.float32)

    full = lambda shape: pl.BlockSpec(shape, lambda i: (0, 0))
    nrow = lambda shape: pl.BlockSpec(
        shape, lambda i: (jnp.minimum(i, NGRID - 1), 0))
    return pl.pallas_call(
        body,
        grid=(EGRID,),
        in_specs=[nrow((RB, D_IN)), full((D_IN, H)), full((1, H)),
                  full((H, H)), full((1, H)), full((H, H)), full((H, 1)),
                  pl.BlockSpec((EB, EDGE_DIM), lambda i: (i, 0)),
                  full((EDGE_DIM, H))],
        out_specs=[nrow((RB, 2 * H)), nrow((RB, H)), nrow((RB, 1)),
                   pl.BlockSpec((EB, TW0), lambda i: (i, 0))],
        out_shape=[jax.ShapeDtypeStruct((NP, 2 * H), jnp.float32),
                   jax.ShapeDtypeStruct((NP, H), jnp.float32),
                   jax.ShapeDtypeStruct((NP, 1), jnp.float32),
                   jax.ShapeDtypeStruct((E, TW0), jnp.float32)],
    )(x, W_pm, b_pm, W_lin1, b_lin1, Ag, gr, ea, Bg)


def _gru_block(hx, hh, Wih, Whh, bih, bhh):
    gi = jnp.dot(hx, Wih, preferred_element_type=jnp.float32) + bih
    gh = jnp.dot(hh, Whh, preferred_element_type=jnp.float32) + bhh
    ir, iz, inn = gi[:, :H], gi[:, H:2 * H], gi[:, 2 * H:]
    hr, hz, hn = gh[:, :H], gh[:, H:2 * H], gh[:, 2 * H:]
    r = jax.nn.sigmoid(ir + hr)
    z = jax.nn.sigmoid(iz + hz)
    nb = jnp.tanh(inn + r * hn)
    return (1.0 - z) * nb + z * hh


def _elu(x):
    return jnp.where(x > 0, x, jnp.exp(jnp.minimum(x, 0.0)) - 1.0)


def _tc_c(accb, x2, g_lin2, g_bias, Wih, Whh, bih, bhh,
          gat_W, att_s, att_d):
    def body(a_r, x2_r, gl2_r, gb_r, wih_r, whh_r, bih_r, bhh_r,
             gw_r, ats_r, atd_r, x3_r, xp_r, as_r, ad_r):
        accf = a_r[0] + a_r[1]
        s = accf[:, H:H + 1]
        hin = accf[:, :H] / (s + 1e-16)
        h = _elu(jnp.dot(hin, gl2_r[...],
                         preferred_element_type=jnp.float32) + gb_r[...])
        x3 = jnp.maximum(
            _gru_block(h, x2_r[...], wih_r[...], whh_r[...], bih_r[...],
                       bhh_r[...]), 0.0)
        x3_r[...] = x3
        xp = jnp.dot(x3, gw_r[...], preferred_element_type=jnp.float32)
        xp_r[...] = xp
        as_r[...] = jnp.dot(xp, ats_r[...],
                            preferred_element_type=jnp.float32)
        ad_r[...] = jnp.dot(xp, atd_r[...],
                            preferred_element_type=jnp.float32)

    full = lambda shape: pl.BlockSpec(shape, lambda i: (0, 0))
    row = lambda shape: pl.BlockSpec(shape, lambda i: (i, 0))
    return pl.pallas_call(
        body,
        grid=(NGRID,),
        in_specs=[pl.BlockSpec((NC, RB, AW), lambda i: (0, i, 0)),
                  row((RB, H)),
                  full((H, H)), full((1, H)), full((H, 3 * H)),
                  full((H, 3 * H)), full((1, 3 * H)), full((1, 3 * H)),
                  full((H, H)), full((H, 1)), full((H, 1))],
        out_specs=[row((RB, H)), row((RB, H)), row((RB, 1)),
                   row((RB, 1))],
        out_shape=[jax.ShapeDtypeStruct((NP, H), jnp.float32),
                   jax.ShapeDtypeStruct((NP, H), jnp.float32),
                   jax.ShapeDtypeStruct((NP, 1), jnp.float32),
                   jax.ShapeDtypeStruct((NP, 1), jnp.float32)],
    )(accb, x2, g_lin2, g_bias, Wih, Whh, bih, bhh, gat_W,
      att_s, att_d)


def _tc_e(accb, x3, batchf, gat_bias, Wih, Whh, bih, bhh,
          W_lin2, b_lin2):
    def body(a_r, x3_r, b_r, gb_r, wih_r, whh_r, bih_r, bhh_r,
             wl2_r, bl2_r, out_r):
        accf = a_r[0] + a_r[1]
        s = accf[:, H:H + 1]
        h2 = _elu(accf[:, :H] / (s + 1e-16) + gb_r[...])
        x4 = jnp.maximum(
            _gru_block(h2, x3_r[...], wih_r[...], whh_r[...], bih_r[...],
                       bhh_r[...]), 0.0)
        node = jnp.dot(x4, wl2_r[...],
                       preferred_element_type=jnp.float32) + bl2_r[...]
        gid = lax.broadcasted_iota(jnp.int32, (RB, NUM_GRAPHS),
                                   1).astype(jnp.float32)
        onehot = (b_r[...] == gid).astype(jnp.float32)
        contrib = lax.dot_general(onehot, node, (((0,), (0,)), ((), ())),
                                  preferred_element_type=jnp.float32)

        @pl.when(pl.program_id(0) == 0)
        def _():
            out_r[...] = jnp.zeros((NUM_GRAPHS, OUT), jnp.float32)

        out_r[...] += contrib

    full = lambda shape: pl.BlockSpec(shape, lambda i: (0, 0))
    row = lambda shape: pl.BlockSpec(shape, lambda i: (i, 0))
    return pl.pallas_call(
        body,
        grid=(NGRID,),
        in_specs=[pl.BlockSpec((NC, RB, AW), lambda i: (0, i, 0)),
                  row((RB, H)), row((RB, 1)),
                  full((1, H)), full((H, 3 * H)), full((H, 3 * H)),
                  full((1, 3 * H)), full((1, 3 * H)), full((H, OUT)),
                  full((1, OUT))],
        out_specs=pl.BlockSpec((NUM_GRAPHS, OUT), lambda i: (0, 0)),
        out_shape=jax.ShapeDtypeStruct((NUM_GRAPHS, OUT), jnp.float32),
    )(accb, x3, batchf, gat_bias, Wih, Whh, bih, bhh, W_lin2, b_lin2)


def kernel(x, edge_index, edge_attr, batch, W_pm, b_pm, W_lin1, b_lin1,
           g_lin1, g_lin2, g_att_l, g_att_r, g_bias, gru0_Wih, gru0_Whh,
           gru0_bih, gru0_bhh, gat_W, gat_att_src, gat_att_dst, gat_bias,
           gru1_Wih, gru1_Whh, gru1_bih, gru1_bhh, W_lin2, b_lin2):
    src = edge_index[0]
    dst = edge_index[1]
    gl = g_att_l
    A = g_lin1[:H]
    B = g_lin1[H:]
    Ag = A * gl[None, :]
    Bg = B * gl[None, :]
    sgn = jnp.sign(gl)

    xpad = jnp.pad(x, ((0, NP - N), (0, 0)))
    T, x2, r, Qp = _tc_a(xpad, W_pm, b_pm.reshape(1, H), W_lin1,
                         b_lin1.reshape(1, H), Ag, g_att_r.reshape(H, 1),
                         edge_attr, Bg)

    acc = _sc_edge_pass(True)(src, dst, Qp, T, r.reshape(NP), sgn)

    x3, xp, asrc, adst = _tc_c(acc, x2, g_lin2,
                               g_bias.reshape(1, H), gru0_Wih, gru0_Whh,
                               gru0_bih.reshape(1, 3 * H),
                               gru0_bhh.reshape(1, 3 * H), gat_W,
                               gat_att_src.reshape(H, 1),
                               gat_att_dst.reshape(H, 1))

    acc2 = _sc_edge_pass(False)(src, dst, xp, asrc.reshape(NP),
                                adst.reshape(NP))

    batchf = jnp.pad(batch.astype(jnp.float32), (0, NP - N),
                     constant_values=-1.0).reshape(NP, 1)
    graph = _tc_e(acc2, x3, batchf, gat_bias.reshape(1, H),
                  gru1_Wih, gru1_Whh, gru1_bih.reshape(1, 3 * H),
                  gru1_bhh.reshape(1, 3 * H), W_lin2,
                  b_lin2.reshape(1, OUT))
    return graph


# speedup vs baseline: 13.8156x; 1.2954x over previous
import functools

import jax
import jax.numpy as jnp
from jax import lax
from jax.experimental import pallas as pl
from jax.experimental.pallas import tpu as pltpu
from jax.experimental.pallas import tpu_sc as plsc

N = 10000
NP = 10240
E = 320000
D_IN = 128
H = 64
EDGE_DIM = 16
OUT = 64
NUM_GRAPHS = 64

NC = 2
NS = 16
NW = NC * NS
EPW = E // NW
C = 80
NCHUNK = EPW // C
ROWS_PER_SUB = NP // NS
ZROWS = 128
AW = 80
TW = 128


def _sc_params():
    return pltpu.CompilerParams(needs_layout_passes=False,
                                use_tc_tiling_on_sc=False)

@functools.lru_cache(maxsize=None)
def _mesh():
    return plsc.VectorSubcoreMesh(
        core_axis_name="c", subcore_axis_name="s",
        num_cores=NC, num_subcores=NS)


def _leaky(x):
    return jnp.where(x >= 0, x, 0.01 * x)


def _ids16(g):
    return g * 16 + lax.iota(jnp.int32, 16)


def _full16(v):
    return jnp.full((16,), v, jnp.int32)


def _sc_edge_pass(gate: bool):
    C = 80 if gate else 128
    TW = 128 if gate else 64
    UPW = 137
    NCH = -(-EPW // C)
    assert NCH % 2 == 1
    nsem = 10 if gate else 8
    scratch = ([pltpu.VMEM((C,), jnp.int32)] * 6
               + [pltpu.VMEM((C, TW), jnp.float32)] * 2
               + [pltpu.VMEM((C, AW), jnp.float32)] * 2
               + [pltpu.VMEM((NP,), jnp.float32)]
               + ([pltpu.VMEM((C, TW), jnp.float32)] * 2
                  + [pltpu.VMEM((H,), jnp.float32)]
                  if gate else
                  [pltpu.VMEM((NP,), jnp.float32)])
               + [pltpu.VMEM_SHARED((NP, AW), jnp.float32)]
               + [pltpu.SemaphoreType.DMA] * nsem)

    def body(*refs):
        if gate:
            (src_h, dst_h, qm_h, t_h, r_h, sgn_h, out_h,
             srcv0, srcv1, dstv0, dstv1, dsts0, dsts1, rows0, rows1,
             msg0, msg1, tab1, qv0, qv1, sgv, acc,
             semg0, semg1, semis0, semis1, semid0, semid1,
             sems0, sems1, semq0, semq1) = refs
            tab2 = None
        else:
            (src_h, dst_h, t_h, asrc_h, adst_h, out_h,
             srcv0, srcv1, dstv0, dstv1, dsts0, dsts1, rows0, rows1,
             msg0, msg1, tab1, tab2, acc,
             semg0, semg1, semis0, semis1, semid0, semid1,
             sems0, sems1) = refs
            qv0 = qv1 = sgv = semq0 = semq1 = None
        srcv = (srcv0, srcv1)
        dstv = (dstv0, dstv1)
        dsts = (dsts0, dsts1)
        rows = (rows0, rows1)
        msgs = (msg0, msg1)
        qv = (qv0, qv1)
        semg = (semg0, semg1)
        semis = (semis0, semis1)
        semid = (semid0, semid1)
        sems = (sems0, sems1)
        semq = (semq0, semq1)

        core = lax.axis_index("c")
        sub = lax.axis_index("s")
        wid = core * NS + sub

        if gate:
            pltpu.sync_copy(r_h, tab1)
            pltpu.sync_copy(sgn_h, sgv)
        else:
            pltpu.sync_copy(asrc_h, tab1)
            pltpu.sync_copy(adst_h, tab2)

        def zfill(i, _):
            for m in msgs:
                for j in range(AW // 16):
                    m[i, pl.ds(16 * j, 16)] = jnp.zeros((16,), jnp.float32)
            return 0
        lax.fori_loop(0, C, zfill, 0)
        row0 = sub * ROWS_PER_SUB
        nz = -(-ROWS_PER_SUB // C)
        for b in range(nz):
            rz = row0 + min(b * C, ROWS_PER_SUB - C)
            pltpu.sync_copy(msgs[0], acc.at[pl.ds(rz, C), :])

        plsc.subcore_barrier()

        def lbase(c):
            return jnp.minimum(c * C, EPW - C)

        def base_of(c):
            return wid * EPW + lbase(c)

        def issue_rowload(c, s):
            pltpu.async_copy(t_h.at[srcv[s]], rows[s], semg[s])
            if gate:
                pltpu.async_copy(qm_h.at[pl.ds(base_of(c), C), :TW], qv[s],
                                 semq[s])

        def wait_rows(s):
            pltpu.make_async_copy(t_h.at[pl.ds(0, C), :TW], rows[s],
                                  semg[s]).wait()
            if gate:
                pltpu.make_async_copy(qm_h.at[pl.ds(0, C), :TW], qv[s],
                                      semq[s]).wait()

        def wait_idx(s):
            pltpu.make_async_copy(src_h.at[pl.ds(0, C)], srcv[s],
                                  semis[s]).wait()
            pltpu.make_async_copy(dst_h.at[pl.ds(0, C)], dstv[s],
                                  semid[s]).wait()

        def wait_scatter(s):
            pltpu.make_async_copy(out_h.at[0, pl.ds(0, C), :], msgs[s],
                                  sems[s]).wait()

        def compute(c, s):
            msg = msgs[s]
            thresh = c * C - lbase(c)
            iota16 = lax.iota(jnp.int32, 16)

            def group(g, _):
                ids = g * 16 + iota16
                dst16 = plsc.load_gather(dstv[s], [ids])
                if gate:
                    sg = [sgv[pl.ds(16 * k, 16)] for k in range(H // 16)]
                    lgv = jnp.zeros((16,), jnp.float32)
                    for k in range(16):
                        rk = _full16(1) * 0 + (ids[k])
                        acc_p = jnp.zeros((16,), jnp.float32)
                        lin_p = jnp.zeros((16,), jnp.float32)
                        for j in range(H // 16):
                            cj = 16 * j + iota16
                            pj = plsc.load_gather(rows[s], [rk, H + cj])
                            qj = plsc.load_gather(qv[s], [rk, cj])
                            u = pj + qj
                            lin_p = lin_p + u
                            acc_p = acc_p + sg[j] * jnp.abs(u)
                        dk = 0.505 * jnp.sum(lin_p) + 0.495 * jnp.sum(acc_p)
                        lgv = jnp.where(iota16 == k, dk, lgv)
                    r16 = plsc.load_gather(tab1, [dst16])
                    logit = _leaky(lgv + r16)
                else:
                    src16 = plsc.load_gather(srcv[s], [ids])
                    as16 = plsc.load_gather(tab1, [src16])
                    ad16 = plsc.load_gather(tab2, [dst16])
                    logit = _leaky(as16 + ad16)
                e16 = jnp.exp(logit)
                e16 = jnp.where(ids >= thresh, e16, 0.0)
                for k in range(16):
                    rk = _full16(1) * 0 + (ids[k])
                    ek = e16[k]
                    for j in range(H // 16):
                        cj = 16 * j + iota16
                        col = plsc.load_gather(rows[s], [rk, cj])
                        plsc.store_scatter(msg, [rk, cj], ek * col)
                    ecol = jnp.where(iota16 == 0, ek, 0.0)
                    plsc.store_scatter(msg, [rk, H + iota16], ecol)
                dsts[s][pl.ds(g * 16, 16)] = dst16
                return 0

            lax.fori_loop(0, C // 16, group, 0)
            pltpu.async_copy(msg, acc.at[dsts[s]], sems[s], add=True)

        pltpu.sync_copy(src_h.at[pl.ds(base_of(0), C)], srcv[0])
        pltpu.sync_copy(dst_h.at[pl.ds(base_of(0), C)], dstv[0])
        issue_rowload(0, 0)
        pltpu.async_copy(src_h.at[pl.ds(base_of(1), C)], srcv[1], semis[1])
        pltpu.async_copy(dst_h.at[pl.ds(base_of(1), C)], dstv[1], semid[1])

        def section(c, s):
            ns = 1 - s
            wait_rows(s)
            wait_idx(ns)
            issue_rowload(c + 1, ns)
            pltpu.async_copy(src_h.at[pl.ds(base_of(c + 2), C)], srcv[s],
                             semis[s])
            wait_scatter(s)
            compute(c, s)
            pltpu.async_copy(dst_h.at[pl.ds(base_of(c + 2), C)], dstv[s],
                             semid[s])

        def pair(k, _):
            section(2 * k, 0)
            section(2 * k + 1, 1)
            return 0

        for s in (0, 1):
            dv = dsts[s]

            def zidx(i, _):
                dv[pl.ds(i * 16, 16)] = jnp.zeros((16,), jnp.int32)
                return 0
            lax.fori_loop(0, C // 16, zidx, 0)
            pltpu.async_copy(msgs[s], acc.at[dsts[s]], sems[s], add=True)

        lax.fori_loop(0, (NCH - 1) // 2, pair, 0)

        wait_rows(0)
        wait_idx(1)
        wait_scatter(0)
        compute(NCH - 1, 0)
        wait_scatter(0)
        wait_scatter(1)

        plsc.subcore_barrier()
        for b in range(ROWS_PER_SUB // ZROWS):
            r0 = row0 + b * ZROWS
            pltpu.sync_copy(acc.at[pl.ds(r0, ZROWS), :],
                            out_h.at[core, pl.ds(r0, ZROWS), :])

    return pl.kernel(
        body,
        out_type=jax.ShapeDtypeStruct((NC, NP, AW), jnp.float32),
        mesh=_mesh(),
        compiler_params=_sc_params(),
        scratch_types=scratch,
    )


_sc_edge_pass = functools.lru_cache(maxsize=None)(_sc_edge_pass)


RB = 512
NGRID = NP // RB
EB = 512
EGRID = E // EB
TW0 = 128


def _tc_a(x, W_pm, b_pm, W_lin1, b_lin1, Ag, gr, ea, Bg):
    def body(x_r, wpm_r, bpm_r, wl1_r, bl1_r, ag_r, gr_r, ea_r, bg_r,
             t_r, x2_r, r_r, qp_r):
        q = jnp.dot(ea_r[...], bg_r[...],
                    preferred_element_type=jnp.float32)
        qp_r[...] = jnp.concatenate(
            [q, jnp.zeros((EB, TW0 - H), jnp.float32)], axis=1)

        @pl.when(pl.program_id(0) < NGRID)
        def _():
            x1 = jnp.dot(x_r[...], wpm_r[...],
                         preferred_element_type=jnp.float32) + bpm_r[...]
            x2 = _leaky(jnp.dot(x1, wl1_r[...],
                                preferred_element_type=jnp.float32)
                        + bl1_r[...])
            x2_r[...] = x2
            pap = jnp.dot(x2, ag_r[...], preferred_element_type=jnp.float32)
            t_r[...] = jnp.concatenate([x2, pap], axis=1)
            r_r[...] = jnp.dot(x2, gr_r[...],
                               preferred_element_type=jnp.float32)

    full = lambda shape: pl.BlockSpec(shape, lambda i: (0, 0))
    nrow = lambda shape: pl.BlockSpec(
        shape, lambda i: (jnp.minimum(i, NGRID - 1), 0))
    return pl.pallas_call(
        body,
        grid=(EGRID,),
        in_specs=[nrow((RB, D_IN)), full((D_IN, H)), full((1, H)),
                  full((H, H)), full((1, H)), full((H, H)), full((H, 1)),
                  pl.BlockSpec((EB, EDGE_DIM), lambda i: (i, 0)),
                  full((EDGE_DIM, H))],
        out_specs=[nrow((RB, 2 * H)), nrow((RB, H)), nrow((RB, 1)),
                   pl.BlockSpec((EB, TW0), lambda i: (i, 0))],
        out_shape=[jax.ShapeDtypeStruct((NP, 2 * H), jnp.float32),
                   jax.ShapeDtypeStruct((NP, H), jnp.float32),
                   jax.ShapeDtypeStruct((NP, 1), jnp.float32),
                   jax.ShapeDtypeStruct((E, TW0), jnp.float32)],
    )(x, W_pm, b_pm, W_lin1, b_lin1, Ag, gr, ea, Bg)


def _gru_block(hx, hh, Wih, Whh, bih, bhh):
    gi = jnp.dot(hx, Wih, preferred_element_type=jnp.float32) + bih
    gh = jnp.dot(hh, Whh, preferred_element_type=jnp.float32) + bhh
    ir, iz, inn = gi[:, :H], gi[:, H:2 * H], gi[:, 2 * H:]
    hr, hz, hn = gh[:, :H], gh[:, H:2 * H], gh[:, 2 * H:]
    r = jax.nn.sigmoid(ir + hr)
    z = jax.nn.sigmoid(iz + hz)
    nb = jnp.tanh(inn + r * hn)
    return (1.0 - z) * nb + z * hh


def _elu(x):
    return jnp.where(x > 0, x, jnp.exp(jnp.minimum(x, 0.0)) - 1.0)


def _tc_c(accb, x2, g_lin2, g_bias, Wih, Whh, bih, bhh,
          gat_W, att_s, att_d):
    def body(a_r, x2_r, gl2_r, gb_r, wih_r, whh_r, bih_r, bhh_r,
             gw_r, ats_r, atd_r, x3_r, xp_r, as_r, ad_r):
        accf = a_r[0] + a_r[1]
        s = accf[:, H:H + 1]
        hin = accf[:, :H] / (s + 1e-16)
        h = _elu(jnp.dot(hin, gl2_r[...],
                         preferred_element_type=jnp.float32) + gb_r[...])
        x3 = jnp.maximum(
            _gru_block(h, x2_r[...], wih_r[...], whh_r[...], bih_r[...],
                       bhh_r[...]), 0.0)
        x3_r[...] = x3
        xp = jnp.dot(x3, gw_r[...], preferred_element_type=jnp.float32)
        xp_r[...] = xp
        as_r[...] = jnp.dot(xp, ats_r[...],
                            preferred_element_type=jnp.float32)
        ad_r[...] = jnp.dot(xp, atd_r[...],
                            preferred_element_type=jnp.float32)

    full = lambda shape: pl.BlockSpec(shape, lambda i: (0, 0))
    row = lambda shape: pl.BlockSpec(shape, lambda i: (i, 0))
    return pl.pallas_call(
        body,
        grid=(NGRID,),
        in_specs=[pl.BlockSpec((NC, RB, AW), lambda i: (0, i, 0)),
                  row((RB, H)),
                  full((H, H)), full((1, H)), full((H, 3 * H)),
                  full((H, 3 * H)), full((1, 3 * H)), full((1, 3 * H)),
                  full((H, H)), full((H, 1)), full((H, 1))],
        out_specs=[row((RB, H)), row((RB, H)), row((RB, 1)),
                   row((RB, 1))],
        out_shape=[jax.ShapeDtypeStruct((NP, H), jnp.float32),
                   jax.ShapeDtypeStruct((NP, H), jnp.float32),
                   jax.ShapeDtypeStruct((NP, 1), jnp.float32),
                   jax.ShapeDtypeStruct((NP, 1), jnp.float32)],
    )(accb, x2, g_lin2, g_bias, Wih, Whh, bih, bhh, gat_W,
      att_s, att_d)


def _tc_e(accb, x3, batchf, gat_bias, Wih, Whh, bih, bhh,
          W_lin2, b_lin2):
    def body(a_r, x3_r, b_r, gb_r, wih_r, whh_r, bih_r, bhh_r,
             wl2_r, bl2_r, out_r):
        accf = a_r[0] + a_r[1]
        s = accf[:, H:H + 1]
        h2 = _elu(accf[:, :H] / (s + 1e-16) + gb_r[...])
        x4 = jnp.maximum(
            _gru_block(h2, x3_r[...], wih_r[...], whh_r[...], bih_r[...],
                       bhh_r[...]), 0.0)
        node = jnp.dot(x4, wl2_r[...],
                       preferred_element_type=jnp.float32) + bl2_r[...]
        gid = lax.broadcasted_iota(jnp.int32, (RB, NUM_GRAPHS),
                                   1).astype(jnp.float32)
        onehot = (b_r[...] == gid).astype(jnp.float32)
        contrib = lax.dot_general(onehot, node, (((0,), (0,)), ((), ())),
                                  preferred_element_type=jnp.float32)

        @pl.when(pl.program_id(0) == 0)
        def _():
            out_r[...] = jnp.zeros((NUM_GRAPHS, OUT), jnp.float32)

        out_r[...] += contrib

    full = lambda shape: pl.BlockSpec(shape, lambda i: (0, 0))
    row = lambda shape: pl.BlockSpec(shape, lambda i: (i, 0))
    return pl.pallas_call(
        body,
        grid=(NGRID,),
        in_specs=[pl.BlockSpec((NC, RB, AW), lambda i: (0, i, 0)),
                  row((RB, H)), row((RB, 1)),
                  full((1, H)), full((H, 3 * H)), full((H, 3 * H)),
                  full((1, 3 * H)), full((1, 3 * H)), full((H, OUT)),
                  full((1, OUT))],
        out_specs=pl.BlockSpec((NUM_GRAPHS, OUT), lambda i: (0, 0)),
        out_shape=jax.ShapeDtypeStruct((NUM_GRAPHS, OUT), jnp.float32),
    )(accb, x3, batchf, gat_bias, Wih, Whh, bih, bhh, W_lin2, b_lin2)


def kernel(x, edge_index, edge_attr, batch, W_pm, b_pm, W_lin1, b_lin1,
           g_lin1, g_lin2, g_att_l, g_att_r, g_bias, gru0_Wih, gru0_Whh,
           gru0_bih, gru0_bhh, gat_W, gat_att_src, gat_att_dst, gat_bias,
           gru1_Wih, gru1_Whh, gru1_bih, gru1_bhh, W_lin2, b_lin2):
    src = edge_index[0]
    dst = edge_index[1]
    gl = g_att_l
    A = g_lin1[:H]
    B = g_lin1[H:]
    Ag = A * gl[None, :]
    Bg = B * gl[None, :]
    sgn = jnp.sign(gl)

    xpad = jnp.pad(x, ((0, NP - N), (0, 0)))
    T, x2, r, Qp = _tc_a(xpad, W_pm, b_pm.reshape(1, H), W_lin1,
                         b_lin1.reshape(1, H), Ag, g_att_r.reshape(H, 1),
                         edge_attr, Bg)

    acc = _sc_edge_pass(True)(src, dst, Qp, T, r.reshape(NP), sgn)

    x3, xp, asrc, adst = _tc_c(acc, x2, g_lin2,
                               g_bias.reshape(1, H), gru0_Wih, gru0_Whh,
                               gru0_bih.reshape(1, 3 * H),
                               gru0_bhh.reshape(1, 3 * H), gat_W,
                               gat_att_src.reshape(H, 1),
                               gat_att_dst.reshape(H, 1))

    acc2 = _sc_edge_pass(False)(src, dst, xp, asrc.reshape(NP),
                                adst.reshape(NP))

    batchf = jnp.pad(batch.astype(jnp.float32), (0, NP - N),
                     constant_values=-1.0).reshape(NP, 1)
    graph = _tc_e(acc2, x3, batchf, gat_bias.reshape(1, H),
                  gru1_Wih, gru1_Whh, gru1_bih.reshape(1, 3 * H),
                  gru1_bhh.reshape(1, 3 * H), W_lin2,
                  b_lin2.reshape(1, OUT))
    return graph
